# baseline (speedup 1.0000x reference)
import functools

import jax
import jax.numpy as jnp
from jax import lax
from jax.experimental import pallas as pl
from jax.experimental.pallas import tpu as pltpu
from jax.experimental.pallas import tpu_sc as plsc

T = 2048
H = 1024
E = 8
K = 2
INTER = 512
SI = 1024
BM = 256
NBLK = T * K // BM + E
NR = NBLK * BM
MW = NBLK + 1
NC = 2
NS = 16
NW = NC * NS
TPW = T // NW
CH = 16
TBLK = 512


def _router_body(x_ref, wg_ref, b_ref, pos0_ref, pos1_ref, w0_ref, w1_ref,
                 meta_ref):
    x = x_ref[...]
    scores = jax.nn.sigmoid(
        jnp.dot(x, wg_ref[...], preferred_element_type=jnp.float32))
    biased = scores + b_ref[...]
    iota = lax.broadcasted_iota(jnp.int32, (T, E), 1)
    m0 = jnp.max(biased, axis=1, keepdims=True)
    sel0 = jnp.min(jnp.where(biased >= m0, iota, E), axis=1, keepdims=True)
    neg = jnp.where(iota == sel0, -jnp.inf, biased)
    m1 = jnp.max(neg, axis=1, keepdims=True)
    sel1 = jnp.min(jnp.where(neg >= m1, iota, E), axis=1, keepdims=True)
    s0 = jnp.sum(jnp.where(iota == sel0, scores, 0.0), axis=1, keepdims=True)
    s1 = jnp.sum(jnp.where(iota == sel1, scores, 0.0), axis=1, keepdims=True)
    denom = s0 + s1 + 1e-20
    w0_ref[...] = jnp.broadcast_to(s0 / denom, (T, 16))
    w1_ref[...] = jnp.broadcast_to(s1 / denom, (T, 16))

    memb = jnp.logical_or(iota == sel0, iota == sel1).astype(jnp.bfloat16)
    rr = lax.broadcasted_iota(jnp.int32, (T, T), 0)
    cc = lax.broadcasted_iota(jnp.int32, (T, T), 1)
    tri = (rr >= cc).astype(jnp.bfloat16)
    csum = jnp.dot(tri, memb, preferred_element_type=jnp.float32)
    counts = csum[T - 1:T, :]
    cnt_pad = jnp.floor((counts + (BM - 1)) / BM) * BM
    er = lax.broadcasted_iota(jnp.int32, (E, E), 0)
    ec = lax.broadcasted_iota(jnp.int32, (E, E), 1)
    upper = (er < ec).astype(jnp.float32)
    offs = jnp.dot(cnt_pad, upper, preferred_element_type=jnp.float32)

    posf0 = jnp.sum(jnp.where(iota == sel0, offs + csum - 1.0, 0.0),
                    axis=1, keepdims=True)
    posf1 = jnp.sum(jnp.where(iota == sel1, offs + csum - 1.0, 0.0),
                    axis=1, keepdims=True)
    pos0_ref[...] = posf0.astype(jnp.int32)
    pos1_ref[...] = posf1.astype(jnp.int32)

    evec = lax.broadcasted_iota(jnp.int32, (1, E), 1)
    off_s = [jnp.sum(jnp.where(evec == e, offs, 0.0)) for e in range(E)]
    cnt_s = [jnp.sum(jnp.where(evec == e, cnt_pad, 0.0)) for e in range(E)]
    nb_total = ((off_s[E - 1] + cnt_s[E - 1]) / BM).astype(jnp.int32)
    bio = lax.broadcasted_iota(jnp.int32, (1, MW), 1)
    bvals = (bio - 1) * BM
    be = sum(((bvals.astype(jnp.float32) >= off_s[e]).astype(jnp.int32))
             for e in range(E)) - 1
    meta_ref[...] = jnp.where(bio == 0, nb_total, be)


def _run_router(x, W_gate, expert_bias):
    return pl.pallas_call(
        _router_body,
        out_shape=(
            jax.ShapeDtypeStruct((T, 1), jnp.int32),
            jax.ShapeDtypeStruct((T, 1), jnp.int32),
            jax.ShapeDtypeStruct((T, 16), jnp.float32),
            jax.ShapeDtypeStruct((T, 16), jnp.float32),
            jax.ShapeDtypeStruct((1, MW), jnp.int32),
        ),
        in_specs=[
            pl.BlockSpec((T, H), lambda: (0, 0)),
            pl.BlockSpec((H, E), lambda: (0, 0)),
            pl.BlockSpec((1, E), lambda: (0, 0)),
        ],
        out_specs=(
            pl.BlockSpec((T, 1), lambda: (0, 0)),
            pl.BlockSpec((T, 1), lambda: (0, 0)),
            pl.BlockSpec((T, 16), lambda: (0, 0)),
            pl.BlockSpec((T, 16), lambda: (0, 0)),
            pl.BlockSpec((1, MW), lambda: (0, 0)),
        ),
    )(x, W_gate, expert_bias.reshape(1, E))


def _dispatch_body(x_hbm, p0_hbm, p1_hbm, xs_hbm, xrows, p0v, p1v, sem0, sem1):
    wid = lax.axis_index("s") * NC + lax.axis_index("c")
    base = wid * TPW
    pltpu.sync_copy(x_hbm.at[pl.ds(base, TPW)], xrows)
    pltpu.sync_copy(p0_hbm.at[pl.ds(base, TPW)], p0v)
    pltpu.sync_copy(p1_hbm.at[pl.ds(base, TPW)], p1v)
    a = pltpu.async_copy(xrows, xs_hbm.at[p0v], sem0)
    b = pltpu.async_copy(xrows, xs_hbm.at[p1v], sem1)
    a.wait()
    b.wait()


def _run_dispatch(x, pos0, pos1):
    mesh = plsc.VectorSubcoreMesh(core_axis_name="c", subcore_axis_name="s")
    f = functools.partial(
        pl.kernel,
        out_type=jax.ShapeDtypeStruct((NR, H), jnp.float32),
        mesh=mesh,
        scratch_types=[
            pltpu.VMEM((TPW, H), jnp.float32),
            pltpu.VMEM((TPW,), jnp.int32),
            pltpu.VMEM((TPW,), jnp.int32),
            pltpu.SemaphoreType.DMA,
            pltpu.SemaphoreType.DMA,
        ],
    )(_dispatch_body)
    return f(x, pos0, pos1)


def _shared_body(xb_ref, wgs_ref, wus_ref, wds_ref, out_ref):
    xb = xb_ref[...]
    hg = jnp.dot(xb, wgs_ref[...], preferred_element_type=jnp.float32)
    hu = jnp.dot(xb, wus_ref[...], preferred_element_type=jnp.float32)
    mid = (jax.nn.silu(hg) * hu).astype(jnp.bfloat16)
    out_ref[...] = jnp.dot(mid, wds_ref[...], preferred_element_type=jnp.float32)


def _run_shared(xb, Wg_s, Wu_s, Wd_s):
    return pl.pallas_call(
        _shared_body,
        grid=(T // TBLK,),
        out_shape=jax.ShapeDtypeStruct((T, H), jnp.float32),
        in_specs=[
            pl.BlockSpec((TBLK, H), lambda t: (t, 0)),
            pl.BlockSpec((H, SI), lambda t: (0, 0)),
            pl.BlockSpec((H, SI), lambda t: (0, 0)),
            pl.BlockSpec((SI, H), lambda t: (0, 0)),
        ],
        out_specs=pl.BlockSpec((TBLK, H), lambda t: (t, 0)),
    )(xb, Wg_s, Wu_s, Wd_s)


def _grouped_body(m_ref, xs_ref, wg_ref, wu_ref, wd_ref, ys_ref):
    b = pl.program_id(0)

    @pl.when(b < m_ref[0])
    def _():
        xb = xs_ref[...].astype(jnp.bfloat16)
        hg = jnp.dot(xb, wg_ref[0], preferred_element_type=jnp.float32)
        hu = jnp.dot(xb, wu_ref[0], preferred_element_type=jnp.float32)
        mid = (jax.nn.silu(hg) * hu).astype(jnp.bfloat16)
        ys_ref[...] = jnp.dot(mid, wd_ref[0], preferred_element_type=jnp.float32)


def _run_grouped(meta1d, xs, Wgb, Wub, Wdb):
    grid_spec = pltpu.PrefetchScalarGridSpec(
        num_scalar_prefetch=1,
        grid=(NBLK,),
        in_specs=[
            pl.BlockSpec((BM, H), lambda b, m: (b, 0)),
            pl.BlockSpec((1, H, INTER), lambda b, m: (m[b + 1], 0, 0)),
            pl.BlockSpec((1, H, INTER), lambda b, m: (m[b + 1], 0, 0)),
            pl.BlockSpec((1, INTER, H), lambda b, m: (m[b + 1], 0, 0)),
        ],
        out_specs=pl.BlockSpec((BM, H), lambda b, m: (b, 0)),
    )
    return pl.pallas_call(
        _grouped_body,
        grid_spec=grid_spec,
        out_shape=jax.ShapeDtypeStruct((NR, H), jnp.float32),
    )(meta1d, xs, Wgb, Wub, Wdb)


def _combine_body(sh_hbm, ys_hbm, p0_hbm, p1_hbm, w0_hbm, w1_hbm, out_hbm,
                  shb, y0b, y1b, ob, p0v, p1v, w0v, w1v, sem0, sem1):
    wid = lax.axis_index("s") * NC + lax.axis_index("c")
    base = wid * TPW
    for c in range(TPW // CH):
        tb = base + c * CH
        pltpu.sync_copy(p0_hbm.at[pl.ds(tb, CH)], p0v)
        pltpu.sync_copy(p1_hbm.at[pl.ds(tb, CH)], p1v)
        a = pltpu.async_copy(ys_hbm.at[p0v], y0b, sem0)
        b = pltpu.async_copy(ys_hbm.at[p1v], y1b, sem1)
        pltpu.sync_copy(w0_hbm.at[pl.ds(tb, CH)], w0v)
        pltpu.sync_copy(w1_hbm.at[pl.ds(tb, CH)], w1v)
        pltpu.sync_copy(sh_hbm.at[pl.ds(tb, CH)], shb)
        a.wait()
        b.wait()

        def row_fn(r, _):
            wv0 = w0v[r, :]
            wv1 = w1v[r, :]

            def col_fn(j, _):
                sl = pl.ds(j * 16, 16)
                ob[r, sl] = shb[r, sl] + wv0 * y0b[r, sl] + wv1 * y1b[r, sl]
                return 0

            lax.fori_loop(0, H // 16, col_fn, 0)
            return 0

        lax.fori_loop(0, CH, row_fn, 0)
        pltpu.sync_copy(ob, out_hbm.at[pl.ds(tb, CH)])


def _run_combine(sh, ys, pos0, pos1, w0, w1):
    mesh = plsc.VectorSubcoreMesh(core_axis_name="c", subcore_axis_name="s")
    f = functools.partial(
        pl.kernel,
        out_type=jax.ShapeDtypeStruct((T, H), jnp.float32),
        mesh=mesh,
        scratch_types=[
            pltpu.VMEM((CH, H), jnp.float32),
            pltpu.VMEM((CH, H), jnp.float32),
            pltpu.VMEM((CH, H), jnp.float32),
            pltpu.VMEM((CH, H), jnp.float32),
            pltpu.VMEM((CH,), jnp.int32),
            pltpu.VMEM((CH,), jnp.int32),
            pltpu.VMEM((CH, 16), jnp.float32),
            pltpu.VMEM((CH, 16), jnp.float32),
            pltpu.SemaphoreType.DMA,
            pltpu.SemaphoreType.DMA,
        ],
    )(_combine_body)
    return f(sh, ys, pos0, pos1, w0, w1)


def kernel(hidden_states, W_gate, Wg_s, Wu_s, Wd_s, Wg, Wu, Wd, expert_bias):
    b, s, h = hidden_states.shape
    x = hidden_states.reshape(T, H)

    pos0, pos1, w0, w1, meta = _run_router(x, W_gate, expert_bias)
    pos0 = pos0.reshape(T)
    pos1 = pos1.reshape(T)
    meta1d = meta.reshape(MW)

    xs = _run_dispatch(x, pos0, pos1)
    sh = _run_shared(x.astype(jnp.bfloat16), Wg_s.astype(jnp.bfloat16),
                     Wu_s.astype(jnp.bfloat16), Wd_s.astype(jnp.bfloat16))
    ys = _run_grouped(meta1d, xs, Wg.astype(jnp.bfloat16),
                      Wu.astype(jnp.bfloat16), Wd.astype(jnp.bfloat16))
    out = _run_combine(sh, ys, pos0, pos1, w0, w1)
    return out.reshape(b, s, h)
